# baseline (speedup 1.0000x reference)
import jax
import jax.numpy as jnp
from jax.experimental import pallas as pl

N_U = 5000
N_I = 5000
D = 128
E = 320000
L = 2
TEMP = 0.2
LAMBDA_1 = 0.2
LAMBDA_2 = 1e-07
LAMBDA_3 = 1e-05
MLP_COF = 1.0
B = 1024


def _final_combine(a_ref, o_ref):
    o_ref[...] = a_ref[...]


def kernel(uids, iids, pos, neg, edge_index, adj_vals, E_u_0, E_i_0, fuse_w, fuse_b, wv_param, W1, b1, W2, b2, a_u, a_i):
    src, dst = edge_index[0], edge_index[1]

    def spmm_u(vals, Ei):
        return jax.ops.segment_sum(vals[:, None] * Ei[dst], src, num_segments=N_U)

    def spmm_i(vals, Eu):
        return jax.ops.segment_sum(vals[:, None] * Eu[src], dst, num_segments=N_I)

    Eu1 = spmm_u(adj_vals, E_i_0); Ei1 = spmm_i(adj_vals, E_u_0)
    Eu2 = spmm_u(adj_vals, Ei1);  Ei2 = spmm_i(adj_vals, Eu1)
    E_u = E_u_0 + Eu1 + Eu2
    E_i = E_i_0 + Ei1 + Ei2

    W1u, W1i = W1[:D], W1[D:]
    Hu = E_u_0 @ W1u
    Hi = E_i_0 @ W1i
    pu = E_u_0 @ a_u
    pi = E_i_0 @ a_i

    dot_mlp = jax.nn.relu(Hu[src] + Hi[dst] + b1) @ W2[:, 0]
    dot_gcn = jnp.sum(E_u[src] * E_i[dst], axis=1)
    attsum = pu[src] + pi[dst]

    g_mlp = jax.nn.sigmoid(dot_mlp + b2[0])
    g_wv = jax.nn.sigmoid(wv_param)
    g_gcn = jax.nn.sigmoid(attsum * 0 + dot_gcn)
    g_att = jax.nn.sigmoid(attsum)
    gs = [g_mlp, g_wv, g_gcn, g_att]
    ts = [jnp.tanh(fuse_w * g + fuse_b) for g in gs]
    S = [jnp.sum(jnp.exp(t)) for t in ts]
    Ag_soft = sum(jnp.exp(t) / s * g for t, s, g in zip(ts, S, gs))
    Ag = (MLP_COF * g_mlp + g_wv + g_gcn + g_att - (MLP_COF + 2.0) * Ag_soft) / (4.0 + MLP_COF)
    baw = g_gcn * Ag
    aug_vals = baw * adj_vals

    Zu1 = spmm_u(aug_vals, E_i_0); Zi1 = spmm_i(aug_vals, E_u_0)
    Zu2 = spmm_u(aug_vals, Zi1);  Zi2 = spmm_i(aug_vals, Zu1)
    Z_u = E_u_0 + Zu1 + Zu2
    Z_i = E_i_0 + Zi1 + Zi2

    neg_score = jnp.log(jnp.sum(jnp.exp(Z_u[uids] @ E_u.T / TEMP), axis=1) + 1e-08).mean()
    neg_score += jnp.log(jnp.sum(jnp.exp(Z_i[iids] @ E_i.T / TEMP), axis=1) + 1e-08).mean()
    pos_score = jnp.clip(jnp.sum(Z_u[uids] * E_u[uids], axis=1) / TEMP, -5.0, 5.0).mean() \
        + jnp.clip(jnp.sum(Z_i[iids] * E_i[iids], axis=1) / TEMP, -5.0, 5.0).mean()
    loss_cl = -pos_score + neg_score
    u_emb, pos_emb, neg_emb = E_u[uids], E_i[pos], E_i[neg]
    pos_scores = jnp.sum(u_emb * pos_emb, axis=-1)
    neg_scores = jnp.sum(u_emb * neg_emb, axis=-1)
    loss_bpr = -jnp.log(jax.nn.sigmoid(pos_scores - neg_scores)).mean()
    loss_pr = LAMBDA_2 * (-jnp.log(baw)).mean()
    params = [E_u_0, E_i_0, fuse_w, fuse_b, wv_param, W1, b1, W2, b2, a_u, a_i]
    loss_reg = sum(jnp.sum(p * p) for p in params) * LAMBDA_3
    loss = loss_bpr + LAMBDA_1 * loss_cl + loss_pr + loss_reg

    vec = jnp.stack([loss, loss_bpr, LAMBDA_1 * loss_cl, loss_pr]).reshape(1, 4)
    out = pl.pallas_call(
        _final_combine,
        out_shape=jax.ShapeDtypeStruct((1, 4), jnp.float32),
    )(vec)
    return out[0, 0], out[0, 1], out[0, 2], out[0, 3]


# speedup vs baseline: 1.6425x; 1.6425x over previous
import functools

import jax
import jax.numpy as jnp
from jax import lax
from jax.experimental import pallas as pl
from jax.experimental.pallas import tpu as pltpu
from jax.experimental.pallas import tpu_sc as plsc

N_U = 5000
N_I = 5000
NPAD = 5120
D = 128
E = 320000
TEMP = 0.2
LAMBDA_1 = 0.2
LAMBDA_2 = 1e-07
LAMBDA_3 = 1e-05
MLP_COF = 1.0
B = 1024

NC = 2
NS = 16
NW = NC * NS
EPW = E // NW
C = 80
NCH = EPW // C
RPT = NPAD // NS
G = 25

_f32 = jnp.float32
_i32 = jnp.int32


def _spmm_body(scaled, ti, tu, src3, dst3, w2, outu, outi,
               src_v, dst_v, w_v, rows_u, rows_i, zbuf, acc_u, acc_i,
               sem_u, sem_i):
    c = lax.axis_index("c")
    s = lax.axis_index("s")
    wid = s * NC + c

    zero = jnp.zeros((16,), _f32)
    for r in range(8):
        for k in range(8):
            zbuf[r, pl.ds(k * 16, 16)] = zero
    for b in range(RPT // 8):
        pltpu.sync_copy(zbuf, acc_u.at[pl.ds(s * RPT + b * 8, 8)])
        pltpu.sync_copy(zbuf, acc_i.at[pl.ds(s * RPT + b * 8, 8)])
    plsc.subcore_barrier()

    for g in range(NCH // G):
        pltpu.sync_copy(src3.at[wid, g], src_v)
        pltpu.sync_copy(dst3.at[wid, g], dst_v)
        if scaled:
            pltpu.sync_copy(w2.at[pl.ds(wid * EPW + g * G * C, G * C)], w_v)

        def chunk(j, carry):
            pltpu.async_copy(ti.at[dst_v.at[j]], rows_u, sem_u).wait()
            pltpu.async_copy(tu.at[src_v.at[j]], rows_i, sem_i).wait()
            if scaled:
                def edge(e, cc):
                    wsp = plsc.load_gather(w_v, [jnp.full((16,), j * C + e, _i32)])
                    for k in range(8):
                        rows_u[e, pl.ds(k * 16, 16)] = rows_u[e, pl.ds(k * 16, 16)] * wsp
                        rows_i[e, pl.ds(k * 16, 16)] = rows_i[e, pl.ds(k * 16, 16)] * wsp
                    return cc
                lax.fori_loop(0, C, edge, 0)
            pltpu.sync_copy(rows_u, acc_u.at[src_v.at[j]], add=True)
            pltpu.sync_copy(rows_i, acc_i.at[dst_v.at[j]], add=True)
            return carry

        lax.fori_loop(0, G, chunk, 0)
    plsc.subcore_barrier()

    pltpu.sync_copy(acc_u.at[pl.ds(s * RPT, RPT)], outu.at[c, pl.ds(s * RPT, RPT)])
    pltpu.sync_copy(acc_i.at[pl.ds(s * RPT, RPT)], outi.at[c, pl.ds(s * RPT, RPT)])


def _make_spmm(scaled):
    mesh = plsc.VectorSubcoreMesh(core_axis_name="c", subcore_axis_name="s")
    scratch = [
        pltpu.VMEM((G, C), _i32),
        pltpu.VMEM((G, C), _i32),
        pltpu.VMEM((G * C,), _f32),
        pltpu.VMEM((C, D), _f32),
        pltpu.VMEM((C, D), _f32),
        pltpu.VMEM((8, D), _f32),
        pltpu.VMEM_SHARED((NPAD, D), _f32),
        pltpu.VMEM_SHARED((NPAD, D), _f32),
        pltpu.SemaphoreType.DMA,
        pltpu.SemaphoreType.DMA,
    ]
    return pl.kernel(
        functools.partial(_spmm_body, scaled),
        out_type=(jax.ShapeDtypeStruct((NC, NPAD, D), _f32),
                  jax.ShapeDtypeStruct((NC, NPAD, D), _f32)),
        mesh=mesh,
        scratch_types=scratch,
        compiler_params=pltpu.CompilerParams(needs_layout_passes=False),
    )


_spmm_plain = _make_spmm(False)
_spmm_scaled = _make_spmm(True)


def _spmm_pair(ti, tu, src3, dst3, w2, scaled):
    f = _spmm_scaled if scaled else _spmm_plain
    pu, pi = f(ti, tu, src3, dst3, w2)
    return pu[0] + pu[1], pi[0] + pi[1]


def kernel(uids, iids, pos, neg, edge_index, adj_vals, E_u_0, E_i_0, fuse_w,
           fuse_b, wv_param, W1, b1, W2, b2, a_u, a_i):
    src = edge_index[0].astype(_i32)
    dst = edge_index[1].astype(_i32)
    src3 = src.reshape(NW, NCH // G, G, C)
    dst3 = dst.reshape(NW, NCH // G, G, C)


    Eu0p = jnp.zeros((NPAD, D), _f32).at[:N_U].set(E_u_0)
    Ei0p = jnp.zeros((NPAD, D), _f32).at[:N_I].set(E_i_0)

    Eu1, Ei1 = _spmm_pair(Ei0p[:N_I], Eu0p[:N_U], src3, dst3, adj_vals, True)
    Eu2, Ei2 = _spmm_pair(Ei1[:N_I], Eu1[:N_U], src3, dst3, adj_vals, True)
    E_u = Eu0p + Eu1 + Eu2
    E_i = Ei0p + Ei1 + Ei2

    W1u, W1i = W1[:D], W1[D:]
    Hu = E_u_0 @ W1u
    Hi = E_i_0 @ W1i
    pu = E_u_0 @ a_u
    pi = E_i_0 @ a_i

    dot_mlp = jax.nn.relu(Hu[src] + Hi[dst] + b1) @ W2[:, 0]
    dot_gcn = jnp.sum(E_u[:N_U][src] * E_i[:N_I][dst], axis=1)
    attsum = pu[src] + pi[dst]

    g_mlp = jax.nn.sigmoid(dot_mlp + b2[0])
    g_wv = jax.nn.sigmoid(wv_param)
    g_gcn = jax.nn.sigmoid(dot_gcn)
    g_att = jax.nn.sigmoid(attsum)
    gs = [g_mlp, g_wv, g_gcn, g_att]
    ts = [jnp.tanh(fuse_w * g + fuse_b) for g in gs]
    S = [jnp.sum(jnp.exp(t)) for t in ts]
    Ag_soft = sum(jnp.exp(t) / sk * g for t, sk, g in zip(ts, S, gs))
    Ag = (MLP_COF * g_mlp + g_wv + g_gcn + g_att - (MLP_COF + 2.0) * Ag_soft) / (4.0 + MLP_COF)
    baw = g_gcn * Ag
    aug_vals = baw * adj_vals


    Zu1, Zi1 = _spmm_pair(Ei0p[:N_I], Eu0p[:N_U], src3, dst3, aug_vals, True)
    Zu2, Zi2 = _spmm_pair(Zi1[:N_I], Zu1[:N_U], src3, dst3, aug_vals, True)
    Z_u = Eu0p + Zu1 + Zu2
    Z_i = Ei0p + Zi1 + Zi2
    Z_u = Z_u[:N_U]
    Z_i = Z_i[:N_I]
    E_u = E_u[:N_U]
    E_i = E_i[:N_I]

    neg_score = jnp.log(jnp.sum(jnp.exp(Z_u[uids] @ E_u.T / TEMP), axis=1) + 1e-08).mean()
    neg_score += jnp.log(jnp.sum(jnp.exp(Z_i[iids] @ E_i.T / TEMP), axis=1) + 1e-08).mean()
    pos_score = jnp.clip(jnp.sum(Z_u[uids] * E_u[uids], axis=1) / TEMP, -5.0, 5.0).mean() \
        + jnp.clip(jnp.sum(Z_i[iids] * E_i[iids], axis=1) / TEMP, -5.0, 5.0).mean()
    loss_cl = -pos_score + neg_score
    u_emb, pos_emb, neg_emb = E_u[uids], E_i[pos], E_i[neg]
    pos_scores = jnp.sum(u_emb * pos_emb, axis=-1)
    neg_scores = jnp.sum(u_emb * neg_emb, axis=-1)
    loss_bpr = -jnp.log(jax.nn.sigmoid(pos_scores - neg_scores)).mean()
    loss_pr = LAMBDA_2 * (-jnp.log(baw)).mean()
    params = [E_u_0, E_i_0, fuse_w, fuse_b, wv_param, W1, b1, W2, b2, a_u, a_i]
    loss_reg = sum(jnp.sum(p * p) for p in params) * LAMBDA_3
    loss = loss_bpr + LAMBDA_1 * loss_cl + loss_pr + loss_reg
    return loss, loss_bpr, LAMBDA_1 * loss_cl, loss_pr


# speedup vs baseline: 4.5856x; 2.7919x over previous
import functools

import jax
import jax.numpy as jnp
from jax import lax
from jax.experimental import pallas as pl
from jax.experimental.pallas import tpu as pltpu
from jax.experimental.pallas import tpu_sc as plsc

N_U = 5000
N_I = 5000
NPAD = 5120
D = 128
E = 320000
TEMP = 0.2
LAMBDA_1 = 0.2
LAMBDA_2 = 1e-07
LAMBDA_3 = 1e-05
MLP_COF = 1.0
B = 1024

NC = 2
NS = 16
NW = NC * NS
EPW = E // NW
C = 80
NCH = EPW // C
RPT = NPAD // NS
G = 25

_f32 = jnp.float32
_i32 = jnp.int32


def _spmm_body(scaled, ti, tu, src3, dst3, w2, outu, outi,
               src_v, dst_v, w_v, rows_u, rows_i, zbuf, acc_u, acc_i,
               sem_u, sem_i):
    c = lax.axis_index("c")
    s = lax.axis_index("s")
    wid = s * NC + c

    zero = jnp.zeros((16,), _f32)
    for r in range(8):
        for k in range(8):
            zbuf[r, pl.ds(k * 16, 16)] = zero
    for b in range(RPT // 8):
        pltpu.sync_copy(zbuf, acc_u.at[pl.ds(s * RPT + b * 8, 8)])
        pltpu.sync_copy(zbuf, acc_i.at[pl.ds(s * RPT + b * 8, 8)])
    plsc.subcore_barrier()

    for g in range(NCH // G):
        pltpu.sync_copy(src3.at[wid, g], src_v)
        pltpu.sync_copy(dst3.at[wid, g], dst_v)
        if scaled:
            pltpu.sync_copy(w2.at[pl.ds(wid * EPW + g * G * C, G * C)], w_v)

        def chunk(j, carry):
            pltpu.async_copy(ti.at[dst_v.at[j]], rows_u, sem_u).wait()
            pltpu.async_copy(tu.at[src_v.at[j]], rows_i, sem_i).wait()
            if scaled:
                def edge(e, cc):
                    wsp = plsc.load_gather(w_v, [jnp.full((16,), j * C + e, _i32)])
                    for k in range(8):
                        rows_u[e, pl.ds(k * 16, 16)] = rows_u[e, pl.ds(k * 16, 16)] * wsp
                        rows_i[e, pl.ds(k * 16, 16)] = rows_i[e, pl.ds(k * 16, 16)] * wsp
                    return cc
                lax.fori_loop(0, C, edge, 0)
            pltpu.sync_copy(rows_u, acc_u.at[src_v.at[j]], add=True)
            pltpu.sync_copy(rows_i, acc_i.at[dst_v.at[j]], add=True)
            return carry

        lax.fori_loop(0, G, chunk, 0)
    plsc.subcore_barrier()

    pltpu.sync_copy(acc_u.at[pl.ds(s * RPT, RPT)], outu.at[c, pl.ds(s * RPT, RPT)])
    pltpu.sync_copy(acc_i.at[pl.ds(s * RPT, RPT)], outi.at[c, pl.ds(s * RPT, RPT)])


def _make_spmm(scaled):
    mesh = plsc.VectorSubcoreMesh(core_axis_name="c", subcore_axis_name="s")
    scratch = [
        pltpu.VMEM((G, C), _i32),
        pltpu.VMEM((G, C), _i32),
        pltpu.VMEM((G * C,), _f32),
        pltpu.VMEM((C, D), _f32),
        pltpu.VMEM((C, D), _f32),
        pltpu.VMEM((8, D), _f32),
        pltpu.VMEM_SHARED((NPAD, D), _f32),
        pltpu.VMEM_SHARED((NPAD, D), _f32),
        pltpu.SemaphoreType.DMA,
        pltpu.SemaphoreType.DMA,
    ]
    return pl.kernel(
        functools.partial(_spmm_body, scaled),
        out_type=(jax.ShapeDtypeStruct((NC, NPAD, D), _f32),
                  jax.ShapeDtypeStruct((NC, NPAD, D), _f32)),
        mesh=mesh,
        scratch_types=scratch,
        compiler_params=pltpu.CompilerParams(needs_layout_passes=False),
    )


_spmm_plain = _make_spmm(False)
_spmm_scaled = _make_spmm(True)


def _views_body(eu_t, ei_t, hu_t, hi_t, bw_t, pu_t, pi_t, src3, dst3,
                dgcn_o, dmlp_o, att_o,
                src_v, dst_v, eu_r, ei_r, hu_r, hi_r, bw_v, pu_v, pi_v,
                dg_v, dm_v, at_v, sem_a, sem_b, sem_c, sem_d):
    c = lax.axis_index("c")
    s = lax.axis_index("s")
    wid = s * NC + c

    pltpu.sync_copy(bw_t, bw_v)
    pltpu.sync_copy(pu_t, pu_v)
    pltpu.sync_copy(pi_t, pi_v)

    for g in range(NCH // G):
        pltpu.sync_copy(src3.at[wid, g], src_v)
        pltpu.sync_copy(dst3.at[wid, g], dst_v)

        def chunk(j, carry):
            pltpu.async_copy(eu_t.at[src_v.at[j]], eu_r, sem_a).wait()
            pltpu.async_copy(ei_t.at[dst_v.at[j]], ei_r, sem_b).wait()
            pltpu.async_copy(hu_t.at[src_v.at[j]], hu_r, sem_c).wait()
            pltpu.async_copy(hi_t.at[dst_v.at[j]], hi_r, sem_d).wait()

            def edge(e, cc):
                accg = jnp.zeros((16,), _f32)
                accm = jnp.zeros((16,), _f32)
                for k in range(8):
                    sl = pl.ds(k * 16, 16)
                    accg = accg + eu_r[e, sl] * ei_r[e, sl]
                    h = jnp.maximum(hu_r[e, sl] + hi_r[e, sl] + bw_v[0, sl], 0.0)
                    accm = accm + h * bw_v[1, sl]
                lane0 = lax.iota(_i32, 16) == 0
                eidx = jnp.full((16,), j * C + e, _i32)
                plsc.store_scatter(dg_v, [eidx],
                                   jnp.full((16,), jnp.sum(accg, axis=0), _f32),
                                   mask=lane0)
                plsc.store_scatter(dm_v, [eidx],
                                   jnp.full((16,), jnp.sum(accm, axis=0), _f32),
                                   mask=lane0)
                return cc
            lax.fori_loop(0, C, edge, 0)

            def att16(t, cc):
                si = src_v[j, pl.ds(t * 16, 16)]
                di = dst_v[j, pl.ds(t * 16, 16)]
                a = plsc.load_gather(pu_v, [si]) + plsc.load_gather(pi_v, [di])
                at_v[pl.ds(j * C + t * 16, 16)] = a
                return cc
            lax.fori_loop(0, C // 16, att16, 0)
            return carry

        lax.fori_loop(0, G, chunk, 0)
        base = wid * EPW + g * G * C
        pltpu.sync_copy(dg_v, dgcn_o.at[pl.ds(base, G * C)])
        pltpu.sync_copy(dm_v, dmlp_o.at[pl.ds(base, G * C)])
        pltpu.sync_copy(at_v, att_o.at[pl.ds(base, G * C)])


def _make_views():
    mesh = plsc.VectorSubcoreMesh(core_axis_name="c", subcore_axis_name="s")
    scratch = [
        pltpu.VMEM((G, C), _i32),
        pltpu.VMEM((G, C), _i32),
        pltpu.VMEM((C, D), _f32),
        pltpu.VMEM((C, D), _f32),
        pltpu.VMEM((C, D), _f32),
        pltpu.VMEM((C, D), _f32),
        pltpu.VMEM((2, D), _f32),
        pltpu.VMEM((N_U,), _f32),
        pltpu.VMEM((N_I,), _f32),
        pltpu.VMEM((G * C,), _f32),
        pltpu.VMEM((G * C,), _f32),
        pltpu.VMEM((G * C,), _f32),
        pltpu.SemaphoreType.DMA,
        pltpu.SemaphoreType.DMA,
        pltpu.SemaphoreType.DMA,
        pltpu.SemaphoreType.DMA,
    ]
    return pl.kernel(
        _views_body,
        out_type=(jax.ShapeDtypeStruct((E,), _f32),
                  jax.ShapeDtypeStruct((E,), _f32),
                  jax.ShapeDtypeStruct((E,), _f32)),
        mesh=mesh,
        scratch_types=scratch,
        compiler_params=pltpu.CompilerParams(needs_layout_passes=False),
    )


_views = _make_views()


def _spmm_pair(ti, tu, src3, dst3, w2, scaled):
    f = _spmm_scaled if scaled else _spmm_plain
    pu, pi = f(ti, tu, src3, dst3, w2)
    return pu[0] + pu[1], pi[0] + pi[1]


def kernel(uids, iids, pos, neg, edge_index, adj_vals, E_u_0, E_i_0, fuse_w,
           fuse_b, wv_param, W1, b1, W2, b2, a_u, a_i):
    src = edge_index[0].astype(_i32)
    dst = edge_index[1].astype(_i32)
    src3 = src.reshape(NW, NCH // G, G, C)
    dst3 = dst.reshape(NW, NCH // G, G, C)


    Eu0p = jnp.zeros((NPAD, D), _f32).at[:N_U].set(E_u_0)
    Ei0p = jnp.zeros((NPAD, D), _f32).at[:N_I].set(E_i_0)

    Eu1, Ei1 = _spmm_pair(Ei0p[:N_I], Eu0p[:N_U], src3, dst3, adj_vals, True)
    Eu2, Ei2 = _spmm_pair(Ei1[:N_I], Eu1[:N_U], src3, dst3, adj_vals, True)
    E_u = Eu0p + Eu1 + Eu2
    E_i = Ei0p + Ei1 + Ei2

    W1u, W1i = W1[:D], W1[D:]
    Hu = E_u_0 @ W1u
    Hi = E_i_0 @ W1i
    pu = E_u_0 @ a_u
    pi = E_i_0 @ a_i

    bw = jnp.stack([b1, W2[:, 0]])
    dot_gcn, dot_mlp, attsum = _views(E_u[:N_U], E_i[:N_I], Hu, Hi, bw, pu, pi,
                                      src3, dst3)

    g_mlp = jax.nn.sigmoid(dot_mlp + b2[0])
    g_wv = jax.nn.sigmoid(wv_param)
    g_gcn = jax.nn.sigmoid(dot_gcn)
    g_att = jax.nn.sigmoid(attsum)
    gs = [g_mlp, g_wv, g_gcn, g_att]
    ts = [jnp.tanh(fuse_w * g + fuse_b) for g in gs]
    S = [jnp.sum(jnp.exp(t)) for t in ts]
    Ag_soft = sum(jnp.exp(t) / sk * g for t, sk, g in zip(ts, S, gs))
    Ag = (MLP_COF * g_mlp + g_wv + g_gcn + g_att - (MLP_COF + 2.0) * Ag_soft) / (4.0 + MLP_COF)
    baw = g_gcn * Ag
    aug_vals = baw * adj_vals


    Zu1, Zi1 = _spmm_pair(Ei0p[:N_I], Eu0p[:N_U], src3, dst3, aug_vals, True)
    Zu2, Zi2 = _spmm_pair(Zi1[:N_I], Zu1[:N_U], src3, dst3, aug_vals, True)
    Z_u = Eu0p + Zu1 + Zu2
    Z_i = Ei0p + Zi1 + Zi2
    Z_u = Z_u[:N_U]
    Z_i = Z_i[:N_I]
    E_u = E_u[:N_U]
    E_i = E_i[:N_I]

    neg_score = jnp.log(jnp.sum(jnp.exp(Z_u[uids] @ E_u.T / TEMP), axis=1) + 1e-08).mean()
    neg_score += jnp.log(jnp.sum(jnp.exp(Z_i[iids] @ E_i.T / TEMP), axis=1) + 1e-08).mean()
    pos_score = jnp.clip(jnp.sum(Z_u[uids] * E_u[uids], axis=1) / TEMP, -5.0, 5.0).mean() \
        + jnp.clip(jnp.sum(Z_i[iids] * E_i[iids], axis=1) / TEMP, -5.0, 5.0).mean()
    loss_cl = -pos_score + neg_score
    u_emb, pos_emb, neg_emb = E_u[uids], E_i[pos], E_i[neg]
    pos_scores = jnp.sum(u_emb * pos_emb, axis=-1)
    neg_scores = jnp.sum(u_emb * neg_emb, axis=-1)
    loss_bpr = -jnp.log(jax.nn.sigmoid(pos_scores - neg_scores)).mean()
    loss_pr = LAMBDA_2 * (-jnp.log(baw)).mean()
    params = [E_u_0, E_i_0, fuse_w, fuse_b, wv_param, W1, b1, W2, b2, a_u, a_i]
    loss_reg = sum(jnp.sum(p * p) for p in params) * LAMBDA_3
    loss = loss_bpr + LAMBDA_1 * loss_cl + loss_pr + loss_reg
    return loss, loss_bpr, LAMBDA_1 * loss_cl, loss_pr
